# Initial kernel scaffold; baseline (speedup 1.0000x reference)
import jax
import jax.numpy as jnp
from jax.experimental import pallas as pl


def kernel(inputs):
    raise NotImplementedError("write your pallas kernel here")


# speedup vs baseline: 7.0532x; 7.0532x over previous
import functools

import jax
import jax.numpy as jnp
from jax import lax
from jax.experimental import pallas as pl
from jax.experimental.pallas import tpu as pltpu
from jax.experimental.pallas import tpu_sc as plsc

P = 128
NUM_CLAUSES = 64
CLAUSE_WEIGHT = 0.5
LANES = 16


def kernel(inputs):
    batch, p = inputs.shape
    info = plsc.get_sparse_core_info()
    nc, ns = info.num_cores, info.num_subcores
    nw = nc * ns
    chunk = 80
    assert batch % chunk == 0
    total_chunks = batch // chunk
    chunks_base = total_chunks // nw
    chunks_rem = total_chunks % nw
    ngrp = NUM_CLAUSES // LANES

    mesh = plsc.VectorSubcoreMesh(core_axis_name="c", subcore_axis_name="s")

    @functools.partial(
        pl.kernel,
        mesh=mesh,
        out_type=jax.ShapeDtypeStruct((batch * p,), jnp.float32),
        compiler_params=pltpu.CompilerParams(needs_layout_passes=False),
        scratch_types=[
            pltpu.VMEM((chunk * p,), jnp.float32),
            pltpu.VMEM((chunk * p,), jnp.float32),
        ],
    )
    def k(in_hbm, out_hbm, x_v, o_v):
        wid = lax.axis_index("s") * nc + lax.axis_index("c")
        lane = jnp.arange(LANES, dtype=jnp.int32)
        cols = []
        for g in range(ngrp):
            ca = (3 * (LANES * g) + 3 * lane) & (p - 1)
            cb = (3 * (LANES * g) + 7 + 3 * lane) & (p - 1)
            cc = (5 * (LANES * g) + 31 + 5 * lane) & (p - 1)
            cols.append((ca, cb, cc))
        zero_v = jnp.zeros((LANES,), jnp.float32)

        n_w = jnp.where(wid < chunks_rem, chunks_base + 1, chunks_base)

        def chunk_body(ci, carry):
            base = (ci * nw + wid) * chunk * p
            pltpu.sync_copy(in_hbm.at[pl.ds(base, chunk * p)], x_v)

            def row_body(r, carry2):
                roff = r * p
                for j in range(p // LANES):
                    o_v[pl.ds(roff + j * LANES, LANES)] = zero_v
                rv = jnp.full((LANES,), roff, jnp.int32)
                for g in range(ngrp):
                    ca, cb, cc = cols[g]
                    fa = rv + ca
                    fb = rv + cb
                    fc = rv + cc
                    va = -plsc.load_gather(x_v, [fa])
                    vb = plsc.load_gather(x_v, [fb])
                    vc = -plsc.load_gather(x_v, [fc])
                    m = jnp.maximum(jnp.maximum(va, vb), vc)
                    ea = jnp.exp(va - m)
                    eb = jnp.exp(vb - m)
                    ec = jnp.exp(vc - m)
                    inv = CLAUSE_WEIGHT / (ea + eb + ec)
                    plsc.addupdate_scatter(o_v, [fa], -(ea * inv))
                    plsc.addupdate_scatter(o_v, [fb], eb * inv)
                    plsc.addupdate_scatter(o_v, [fc], -(ec * inv))
                return carry2

            lax.fori_loop(0, chunk, row_body, 0)
            pltpu.sync_copy(o_v, out_hbm.at[pl.ds(base, chunk * p)])
            return carry

        lax.fori_loop(0, n_w, chunk_body, 0)

    return k(inputs.reshape(batch * p)).reshape(batch, p)


# speedup vs baseline: 12.4634x; 1.7670x over previous
import functools

import jax
import jax.numpy as jnp
from jax import lax
from jax.experimental import pallas as pl
from jax.experimental.pallas import tpu as pltpu
from jax.experimental.pallas import tpu_sc as plsc

P = 128
NUM_CLAUSES = 64
CLAUSE_WEIGHT = 0.5
LANES = 16


def kernel(inputs):
    batch, p = inputs.shape
    info = plsc.get_sparse_core_info()
    nc, ns = info.num_cores, info.num_subcores
    nw = nc * ns
    chunk = 80
    assert batch % chunk == 0
    total_chunks = batch // chunk
    chunks_base = total_chunks // nw
    chunks_rem = total_chunks % nw
    ngrp = NUM_CLAUSES // LANES

    mesh = plsc.VectorSubcoreMesh(core_axis_name="c", subcore_axis_name="s")

    @functools.partial(
        pl.kernel,
        mesh=mesh,
        out_type=jax.ShapeDtypeStruct((batch * p,), jnp.float32),
        compiler_params=pltpu.CompilerParams(needs_layout_passes=False),
        scratch_types=[
            pltpu.VMEM((chunk * p,), jnp.float32),
            pltpu.VMEM((chunk * p,), jnp.float32),
        ],
    )
    def k(in_hbm, out_hbm, x_v, o_v):
        wid = lax.axis_index("s") * nc + lax.axis_index("c")
        lane = jnp.arange(LANES, dtype=jnp.int32)
        cols = []
        for g in range(ngrp):
            ca = (3 * (LANES * g) + 3 * lane) & (p - 1)
            cb = (3 * (LANES * g) + 7 + 3 * lane) & (p - 1)
            cc = (5 * (LANES * g) + 31 + 5 * lane) & (p - 1)
            cols.append((ca, cb, cc))
        zero_v = jnp.zeros((LANES,), jnp.float32)

        n_w = jnp.where(wid < chunks_rem, chunks_base + 1, chunks_base)

        def chunk_body(ci, carry):
            base = (ci * nw + wid) * chunk * p
            pltpu.sync_copy(in_hbm.at[pl.ds(base, chunk * p)], x_v)

            @plsc.parallel_loop(0, chunk, unroll=4)
            def row_body(r):
                roff = r * p
                for j in range(p // LANES):
                    o_v[pl.ds(roff + j * LANES, LANES)] = zero_v
                rv = jnp.full((LANES,), roff, jnp.int32)
                for g in range(ngrp):
                    ca, cb, cc = cols[g]
                    fa = rv + ca
                    fb = rv + cb
                    fc = rv + cc
                    va = -plsc.load_gather(x_v, [fa])
                    vb = plsc.load_gather(x_v, [fb])
                    vc = -plsc.load_gather(x_v, [fc])
                    m = jnp.maximum(jnp.maximum(va, vb), vc)
                    ea = jnp.exp(va - m)
                    eb = jnp.exp(vb - m)
                    ec = jnp.exp(vc - m)
                    inv = CLAUSE_WEIGHT / (ea + eb + ec)
                    plsc.addupdate_scatter(o_v, [fa], -(ea * inv))
                    plsc.addupdate_scatter(o_v, [fb], eb * inv)
                    plsc.addupdate_scatter(o_v, [fc], -(ec * inv))
            pltpu.sync_copy(o_v, out_hbm.at[pl.ds(base, chunk * p)])
            return carry

        lax.fori_loop(0, n_w, chunk_body, 0)

    return k(inputs.reshape(batch * p)).reshape(batch, p)
